# baseline (speedup 1.0000x reference)
import functools

import jax
import jax.numpy as jnp
from jax import lax
from jax.experimental import pallas as pl
from jax.experimental.pallas import tpu as pltpu
from jax.experimental.pallas import tpu_sc as plsc

D_EMBED = 128
N_VOCAB = 101
BATCH = 4096
SEQ = 200
SEQ_PAD = 208
LANES = 16
NCOL = D_EMBED // LANES
N_GROUP = SEQ_PAD // LANES
ROW_W = (SEQ + 1) * D_EMBED
BUF_W = (SEQ_PAD + 1) * D_EMBED


def _sc_embed(x1, tab1, cls1):
    info = plsc.get_sparse_core_info()
    nw = info.num_cores * info.num_subcores
    nb = BATCH // nw
    ntok = nb * SEQ_PAD

    mesh = plsc.VectorSubcoreMesh(core_axis_name="c", subcore_axis_name="s")

    @functools.partial(
        pl.kernel,
        out_type=jax.ShapeDtypeStruct((BATCH, ROW_W), jnp.float32),
        mesh=mesh,
        scratch_types=[
            pltpu.VMEM((N_VOCAB * D_EMBED,), jnp.float32),
            pltpu.VMEM((nb * SEQ_PAD,), jnp.int32),
            pltpu.VMEM((BUF_W,), jnp.float32),
            pltpu.VMEM((BUF_W,), jnp.float32),
            pltpu.SemaphoreType.DMA,
            pltpu.SemaphoreType.DMA,
        ],
    )
    def k(x_hbm, tab_hbm, cls_hbm, out_hbm, tab_v, idx_v, buf_a, buf_b,
          osem_a, osem_b):
        sid = lax.axis_index("s")
        wid = sid * info.num_cores + lax.axis_index("c")

        pltpu.sync_copy(tab_hbm, tab_v)
        pltpu.sync_copy(cls_hbm, buf_a.at[pl.ds(0, D_EMBED)])
        pltpu.sync_copy(cls_hbm, buf_b.at[pl.ds(0, D_EMBED)])
        pltpu.sync_copy(x_hbm.at[pl.ds(wid * ntok, ntok)], idx_v)

        def fill(j, buf):
            def group_body(g, carry):
                base = j * SEQ_PAD + g * LANES
                tok16 = idx_v[pl.ds(base, LANES)] * D_EMBED
                dst0 = (1 + g * LANES) * D_EMBED
                for l in range(LANES):
                    src = tok16[l]
                    dst = dst0 + l * D_EMBED
                    for kk in range(NCOL):
                        buf[pl.ds(dst + LANES * kk, LANES)] = (
                            tab_v[pl.ds(src + LANES * kk, LANES)])
                return carry

            lax.fori_loop(0, N_GROUP, group_body, 0)

        def put(j, buf, sem):
            return pltpu.async_copy(buf.at[pl.ds(0, ROW_W)],
                                    out_hbm.at[wid * nb + j], sem)

        fill(0, buf_a)

        def body(i, carry):
            wa = put(2 * i, buf_a, osem_a)
            fill(2 * i + 1, buf_b)
            wb = put(2 * i + 1, buf_b, osem_b)
            wa.wait()

            @pl.when(i + 1 < nb // 2)
            def _():
                fill(2 * i + 2, buf_a)
            wb.wait()
            return carry

        lax.fori_loop(0, nb // 2, body, 0)

    return k(x1, tab1, cls1)


def kernel(x, embed, first_cls):
    xp = jnp.pad(x.astype(jnp.int32), ((0, 0), (0, SEQ_PAD - SEQ)))
    out = _sc_embed(xp.reshape(-1), embed.reshape(-1), first_cls.reshape(-1))
    return out.reshape(BATCH, SEQ + 1, D_EMBED)


# speedup vs baseline: 1.0148x; 1.0148x over previous
import functools

import jax
import jax.numpy as jnp
from jax import lax
from jax.experimental import pallas as pl
from jax.experimental.pallas import tpu as pltpu
from jax.experimental.pallas import tpu_sc as plsc

D_EMBED = 128
N_VOCAB = 101
BATCH = 4096
SEQ = 200
LANES = 16
NCOL = D_EMBED // LANES
ROW_W = (SEQ + 1) * D_EMBED
N_GROUP = -(-SEQ // LANES)


def _sc_embed(x1, tab1, cls1):
    info = plsc.get_sparse_core_info()
    nw = info.num_cores * info.num_subcores
    nb = BATCH // nw

    mesh = plsc.VectorSubcoreMesh(core_axis_name="c", subcore_axis_name="s")

    @functools.partial(
        pl.kernel,
        out_type=jax.ShapeDtypeStruct((BATCH, ROW_W), jnp.float32),
        mesh=mesh,
        scratch_types=[
            pltpu.VMEM((N_VOCAB * D_EMBED,), jnp.float32),
            pltpu.VMEM((ROW_W,), jnp.float32),
            pltpu.VMEM((ROW_W,), jnp.float32),
            pltpu.VMEM((BATCH // 32 * SEQ,), jnp.int32),
            pltpu.SemaphoreType.DMA,
            pltpu.SemaphoreType.DMA,
        ],
    )
    def k(x_hbm, tab_hbm, cls_hbm, out_hbm, tab_v, buf_a, buf_b,
          idx_v, osem_a, osem_b):
        sid = lax.axis_index("s")
        wid = sid * info.num_cores + lax.axis_index("c")
        xbase = wid * nb * SEQ

        pltpu.sync_copy(tab_hbm, tab_v)
        pltpu.sync_copy(cls_hbm, buf_a.at[pl.ds(0, D_EMBED)])
        pltpu.sync_copy(cls_hbm, buf_b.at[pl.ds(0, D_EMBED)])
        pltpu.sync_copy(x_hbm.at[pl.ds(xbase, nb * SEQ)], idx_v)

        def fill(j, buf):
            def group_body(g, carry):
                off = lax.min(g * LANES, SEQ - LANES)
                tok16 = idx_v[pl.ds(j * SEQ + off, LANES)] * D_EMBED
                dst0 = (1 + off) * D_EMBED
                for l in range(LANES):
                    row = tok16[l]
                    dst = dst0 + l * D_EMBED
                    for kk in range(NCOL):
                        buf[pl.ds(dst + LANES * kk, LANES)] = (
                            tab_v[pl.ds(row + LANES * kk, LANES)])
                return carry

            lax.fori_loop(0, N_GROUP, group_body, 0)

        def put(j, buf, sem):
            return pltpu.async_copy(buf, out_hbm.at[wid * nb + j], sem)

        fill(0, buf_a)

        def body(i, carry):
            wa = put(2 * i, buf_a, osem_a)
            fill(2 * i + 1, buf_b)
            wb = put(2 * i + 1, buf_b, osem_b)
            wa.wait()

            @pl.when(i + 1 < nb // 2)
            def _():
                fill(2 * i + 2, buf_a)
            wb.wait()
            return carry

        lax.fori_loop(0, nb // 2, body, 0)

    return k(x1, tab1, cls1)


def kernel(x, embed, first_cls):
    out = _sc_embed(x.astype(jnp.int32).reshape(-1), embed.reshape(-1),
                    first_cls.reshape(-1))
    return out.reshape(BATCH, SEQ + 1, D_EMBED)


# speedup vs baseline: 3.1872x; 3.1409x over previous
import functools

import jax
import jax.numpy as jnp
from jax import lax
from jax.experimental import pallas as pl
from jax.experimental.pallas import tpu as pltpu
from jax.experimental.pallas import tpu_sc as plsc

D_EMBED = 128
N_VOCAB = 101
BATCH = 4096
SEQ = 200
CHUNK = 100
N_CHUNK = SEQ // CHUNK


def _sc_embed(x2, embed, cls_row):
    info = plsc.get_sparse_core_info()
    nw = info.num_cores * info.num_subcores
    nb = BATCH // nw

    mesh = plsc.VectorSubcoreMesh(core_axis_name="c", subcore_axis_name="s")

    @functools.partial(
        pl.kernel,
        out_type=jax.ShapeDtypeStruct((BATCH, SEQ + 1, D_EMBED), jnp.float32),
        mesh=mesh,
        scratch_types=[
            pltpu.VMEM_SHARED((N_VOCAB, D_EMBED), jnp.float32),
            pltpu.VMEM((nb * N_CHUNK, CHUNK), jnp.int32),
            pltpu.VMEM((SEQ + 1, D_EMBED), jnp.float32),
            pltpu.VMEM((SEQ + 1, D_EMBED), jnp.float32),
            pltpu.SemaphoreType.DMA,
            pltpu.SemaphoreType.DMA,
            pltpu.SemaphoreType.DMA,
        ],
    )
    def k(x_hbm, tab_hbm, cls_hbm, out_hbm, tab_v, idx_v, buf_a, buf_b,
          gsem, osem_a, osem_b):
        sid = lax.axis_index("s")
        wid = sid * info.num_cores + lax.axis_index("c")

        @pl.when(sid == 0)
        def _():
            pltpu.sync_copy(tab_hbm, tab_v)
        pltpu.sync_copy(cls_hbm, buf_a.at[pl.ds(0, 1)])
        pltpu.sync_copy(cls_hbm, buf_b.at[pl.ds(0, 1)])
        plsc.subcore_barrier()
        pltpu.sync_copy(x_hbm.at[pl.ds(wid * (nb * N_CHUNK), nb * N_CHUNK)], idx_v)

        def gather(j, buf):
            cps = []
            for c in range(N_CHUNK):
                cps.append(pltpu.async_copy(
                    tab_v.at[idx_v.at[j * N_CHUNK + c]],
                    buf.at[pl.ds(1 + c * CHUNK, CHUNK)],
                    gsem))
            for cp in cps:
                cp.wait()

        def put(j, buf, sem):
            return pltpu.async_copy(buf, out_hbm.at[wid * nb + j], sem)

        gather(0, buf_a)

        def body(i, carry):
            wa = put(2 * i, buf_a, osem_a)
            gather(2 * i + 1, buf_b)
            wb = put(2 * i + 1, buf_b, osem_b)
            wa.wait()

            @pl.when(i + 1 < nb // 2)
            def _():
                gather(2 * i + 2, buf_a)
            wb.wait()
            return carry

        lax.fori_loop(0, nb // 2, body, 0)

    return k(x2, embed, cls_row)


def kernel(x, embed, first_cls):
    x2 = x.astype(jnp.int32).reshape(BATCH * N_CHUNK, CHUNK)
    cls_row = first_cls.reshape(1, D_EMBED)
    return _sc_embed(x2, embed, cls_row)
